# baseline (speedup 1.0000x reference)
import functools

import jax
import jax.numpy as jnp
from jax import lax
from jax.experimental import pallas as pl
from jax.experimental.pallas import tpu as pltpu
from jax.experimental.pallas import tpu_sc as plsc

B = 16384
D = 64
V_UID = 1000000
V_IID = 100000
NC = 2
NS = 16
NW = NC * NS
BPW = B // NW
CH = 128
NCH = BPW // CH
VEC = 16


def _make_reformat(v, bm):
    half = v // 2
    nblk = half // bm

    def _descs(x, tab_ref, s1, s2, sem1, sem2):
        off = pl.multiple_of(x * bm, 8)
        return (
            pltpu.make_async_copy(tab_ref.at[pl.ds(off, bm)], s1, sem1),
            pltpu.make_async_copy(
                tab_ref.at[pl.ds(half + off, bm)], s2, sem2),
        )

    def _body(tab_ref, o_ref, s1a, s2a, s1b, s2b,
              sem1a, sem2a, sem1b, sem2b):
        i = pl.program_id(0)

        @pl.when(i == 0)
        def _():
            for d in _descs(i, tab_ref, s1a, s2a, sem1a, sem2a):
                d.start()

        @pl.when(i % 2 == 0)
        def _():
            @pl.when(i + 1 < nblk)
            def _():
                for d in _descs(i + 1, tab_ref, s1b, s2b, sem1b, sem2b):
                    d.start()
            for d in _descs(i, tab_ref, s1a, s2a, sem1a, sem2a):
                d.wait()
            o_ref[:, 0:D] = s1a[...]
            o_ref[:, D:2 * D] = s2a[...]

        @pl.when(i % 2 == 1)
        def _():
            @pl.when(i + 1 < nblk)
            def _():
                for d in _descs(i + 1, tab_ref, s1a, s2a, sem1a, sem2a):
                    d.start()
            for d in _descs(i, tab_ref, s1b, s2b, sem1b, sem2b):
                d.wait()
            o_ref[:, 0:D] = s1b[...]
            o_ref[:, D:2 * D] = s2b[...]

    return pl.pallas_call(
        _body,
        grid=(nblk,),
        in_specs=[pl.BlockSpec(memory_space=pl.ANY)],
        out_specs=pl.BlockSpec((bm, 2 * D), lambda i: (i, 0)),
        out_shape=jax.ShapeDtypeStruct((half, 2 * D), jnp.float32),
        scratch_shapes=[
            pltpu.VMEM((bm, D), jnp.float32),
            pltpu.VMEM((bm, D), jnp.float32),
            pltpu.VMEM((bm, D), jnp.float32),
            pltpu.VMEM((bm, D), jnp.float32),
            pltpu.SemaphoreType.DMA,
            pltpu.SemaphoreType.DMA,
            pltpu.SemaphoreType.DMA,
            pltpu.SemaphoreType.DMA,
        ],
    )


_reformat_uid = _make_reformat(V_UID, 5000)
_reformat_iid = _make_reformat(V_IID, 2000)

HALF_U = V_UID // 2
HALF_I = V_IID // 2

_mesh = plsc.VectorSubcoreMesh(core_axis_name="c", subcore_axis_name="s")


@functools.partial(
    pl.kernel,
    mesh=_mesh,
    out_type=jax.ShapeDtypeStruct((B, 2 * D), jnp.float32),
    scratch_types=[
        pltpu.VMEM((NCH, CH), jnp.int32),
        pltpu.VMEM((NCH, CH), jnp.int32),
        pltpu.SMEM((BPW,), jnp.int32),
        pltpu.SMEM((BPW,), jnp.int32),
        pltpu.VMEM((CH, 2 * D), jnp.float32),
        pltpu.VMEM((CH, 2 * D), jnp.float32),
        pltpu.VMEM((CH, 2 * D), jnp.float32),
        pltpu.VMEM((CH, 2 * D), jnp.float32),
        pltpu.VMEM((CH, 2 * D), jnp.float32),
        pltpu.VMEM((CH, 2 * D), jnp.float32),
        pltpu.SemaphoreType.DMA,
        pltpu.SemaphoreType.DMA,
        pltpu.SemaphoreType.DMA,
        pltpu.SemaphoreType.DMA,
    ],
)
def _emb_kernel(uid_hbm, iid_hbm, tu2_hbm, ti2_hbm, out_hbm,
                uidx_v, iidx_v, us_s, is_s,
                ugrp_a, ugrp_b, igrp_a, igrp_b, crows_a, crows_b,
                sem_a, sem_b, sem_wa, sem_wb):
    wid = lax.axis_index("s") * NC + lax.axis_index("c")
    base = wid * BPW

    for j in range(NCH):
        pltpu.sync_copy(uid_hbm.at[pl.ds(base + j * CH, CH)], uidx_v.at[j])
        pltpu.sync_copy(iid_hbm.at[pl.ds(base + j * CH, CH)], iidx_v.at[j])

    def _stage(q, carry):
        j = q >> 3
        t = q & 7
        xu = uidx_v[j, pl.ds(t * VEC, VEC)]
        xi = iidx_v[j, pl.ds(t * VEC, VEC)]
        for l in range(VEC):
            us_s[q * VEC + l] = xu[l]
            is_s[q * VEC + l] = xi[l]
        uidx_v[j, pl.ds(t * VEC, VEC)] = (
            xu - jnp.where(xu >= HALF_U, HALF_U, 0))
        iidx_v[j, pl.ds(t * VEC, VEC)] = (
            xi - jnp.where(xi >= HALF_I, HALF_I, 0))
        return carry

    lax.fori_loop(0, BPW // VEC, _stage, 0)

    def _fire(j, ugrp, igrp, sem):
        pltpu.async_copy(tu2_hbm.at[uidx_v.at[j]], ugrp, sem)
        pltpu.async_copy(ti2_hbm.at[iidx_v.at[j]], igrp, sem)

    def _gwait(j, ugrp, igrp, sem):
        pltpu.make_async_copy(tu2_hbm.at[uidx_v.at[j]], ugrp, sem).wait()
        pltpu.make_async_copy(ti2_hbm.at[iidx_v.at[j]], igrp, sem).wait()

    def _extract(j, ugrp, igrp, crows):
        cbase = j * CH

        def body(r, carry):
            offu = jnp.where(us_s[cbase + r] >= HALF_U, D, 0)
            offi = jnp.where(is_s[cbase + r] >= HALF_I, D, 0)
            for k in range(D // VEC):
                crows[r, pl.ds(k * VEC, VEC)] = (
                    ugrp[r, pl.ds(offu + k * VEC, VEC)])
                crows[r, pl.ds(D + k * VEC, VEC)] = (
                    igrp[r, pl.ds(offi + k * VEC, VEC)])
            return carry

        lax.fori_loop(0, CH, body, 0)

    def _wdesc(crows, sem_w, j):
        return pltpu.make_async_copy(
            crows, out_hbm.at[pl.ds(base + j * CH, CH)], sem_w)

    _fire(0, ugrp_a, igrp_a, sem_a)
    for j in range(NCH):
        even = (j % 2 == 0)
        ugrp = ugrp_a if even else ugrp_b
        igrp = igrp_a if even else igrp_b
        crows = crows_a if even else crows_b
        sem = sem_a if even else sem_b
        sem_w = sem_wa if even else sem_wb
        if j + 1 < NCH:
            _fire(j + 1, ugrp_b if even else ugrp_a,
                  igrp_b if even else igrp_a,
                  sem_b if even else sem_a)
        _gwait(j, ugrp, igrp, sem)
        if j >= 2:
            _wdesc(crows, sem_w, j - 2).wait()
        _extract(j, ugrp, igrp, crows)
        _wdesc(crows, sem_w, j).start()
    _wdesc(crows_a, sem_wa, NCH - 2).wait()
    _wdesc(crows_b, sem_wb, NCH - 1).wait()


def kernel(uid, iid, table_uid, table_iid):
    uid = uid.astype(jnp.int32)
    iid = iid.astype(jnp.int32)
    tu2 = _reformat_uid(table_uid)
    ti2 = _reformat_iid(table_iid)
    return _emb_kernel(uid, iid, tu2, ti2)


# speedup vs baseline: 1.0449x; 1.0449x over previous
import functools

import jax
import jax.numpy as jnp
from jax import lax
from jax.experimental import pallas as pl
from jax.experimental.pallas import tpu as pltpu
from jax.experimental.pallas import tpu_sc as plsc

B = 16384
D = 64
NC = 2
NS = 16
NW = NC * NS
BPW = B // NW
VEC = 16

_mesh = plsc.VectorSubcoreMesh(core_axis_name="c", subcore_axis_name="s")


@functools.partial(
    pl.kernel,
    mesh=_mesh,
    out_type=jax.ShapeDtypeStruct((B, 2 * D), jnp.float32),
    compiler_params=pltpu.CompilerParams(use_tc_tiling_on_sc=False),
    scratch_types=[
        pltpu.VMEM((BPW,), jnp.int32),
        pltpu.VMEM((BPW,), jnp.int32),
        pltpu.SMEM((BPW,), jnp.int32),
        pltpu.SMEM((BPW,), jnp.int32),
        pltpu.VMEM((BPW, 2 * D), jnp.float32),
        pltpu.SemaphoreType.DMA,
        pltpu.SemaphoreType.DMA,
    ],
)
def _emb_kernel(uid_hbm, iid_hbm, tuid_hbm, tiid_hbm, out_hbm,
                uidx_v, iidx_v, us_s, is_s, crows_v, sem_u, sem_i):
    wid = lax.axis_index("s") * NC + lax.axis_index("c")
    base = wid * BPW

    pltpu.sync_copy(uid_hbm.at[pl.ds(base, BPW)], uidx_v)
    pltpu.sync_copy(iid_hbm.at[pl.ds(base, BPW)], iidx_v)

    def _stage(t, carry):
        xu = uidx_v[pl.ds(t * VEC, VEC)]
        xi = iidx_v[pl.ds(t * VEC, VEC)]
        for l in range(VEC):
            us_s[t * VEC + l] = xu[l]
            is_s[t * VEC + l] = xi[l]
        return carry

    lax.fori_loop(0, BPW // VEC, _stage, 0)

    def _fire(r, carry):
        iu = us_s[r]
        ii = is_s[r]
        pltpu.async_copy(tuid_hbm.at[pl.ds(iu, 1)],
                         crows_v.at[pl.ds(r, 1), pl.ds(0, D)], sem_u)
        pltpu.async_copy(tiid_hbm.at[pl.ds(ii, 1)],
                         crows_v.at[pl.ds(r, 1), pl.ds(D, D)], sem_i)
        return carry

    lax.fori_loop(0, BPW, _fire, 0)

    def _drain(r, carry):
        pltpu.make_async_copy(tuid_hbm.at[pl.ds(0, 1)],
                              crows_v.at[pl.ds(0, 1), pl.ds(0, D)],
                              sem_u).wait()
        pltpu.make_async_copy(tiid_hbm.at[pl.ds(0, 1)],
                              crows_v.at[pl.ds(0, 1), pl.ds(D, D)],
                              sem_i).wait()
        return carry

    lax.fori_loop(0, BPW, _drain, 0)

    pltpu.sync_copy(crows_v, out_hbm.at[pl.ds(base, BPW)])


def kernel(uid, iid, table_uid, table_iid):
    uid = uid.astype(jnp.int32)
    iid = iid.astype(jnp.int32)
    return _emb_kernel(uid, iid, table_uid, table_iid)


# speedup vs baseline: 1.5331x; 1.4672x over previous
import functools

import jax
import jax.numpy as jnp
from jax import lax
from jax.experimental import pallas as pl
from jax.experimental.pallas import tpu as pltpu
from jax.experimental.pallas import tpu_sc as plsc

B = 16384
D = 64
NC = 2
NS = 16
NW = NC * NS
BPW = B // NW
K = 16
NCHUNK = BPW // K
VEC = 16

_mesh = plsc.VectorSubcoreMesh(core_axis_name="c", subcore_axis_name="s")


@functools.partial(
    pl.kernel,
    mesh=_mesh,
    out_type=jax.ShapeDtypeStruct((B, 2 * D), jnp.float32),
    scratch_types=[
        pltpu.VMEM((BPW,), jnp.int32),
        pltpu.VMEM((BPW,), jnp.int32),
        pltpu.SMEM((BPW,), jnp.int32),
        pltpu.SMEM((BPW,), jnp.int32),
        pltpu.VMEM((K * 8, D), jnp.float32),
        pltpu.VMEM((K * 8, D), jnp.float32),
        pltpu.VMEM((K * 8, D), jnp.float32),
        pltpu.VMEM((K * 8, D), jnp.float32),
        pltpu.VMEM((K, 2 * D), jnp.float32),
        pltpu.VMEM((K, 2 * D), jnp.float32),
        pltpu.SemaphoreType.DMA,
        pltpu.SemaphoreType.DMA,
        pltpu.SemaphoreType.DMA,
        pltpu.SemaphoreType.DMA,
    ],
)
def _emb_kernel(uid_hbm, iid_hbm, tuid_hbm, tiid_hbm, out_hbm,
                uidx_v, iidx_v, us_s, is_s,
                ugrp_a, ugrp_b, igrp_a, igrp_b, crows_a, crows_b,
                sem_a, sem_b, sem_wa, sem_wb):
    wid = lax.axis_index("s") * NC + lax.axis_index("c")
    base = wid * BPW

    pltpu.sync_copy(uid_hbm.at[pl.ds(base, BPW)], uidx_v)
    pltpu.sync_copy(iid_hbm.at[pl.ds(base, BPW)], iidx_v)

    def _stage(t, carry):
        xu = uidx_v[pl.ds(t * VEC, VEC)]
        xi = iidx_v[pl.ds(t * VEC, VEC)]
        for l in range(VEC):
            us_s[t * VEC + l] = xu[l]
            is_s[t * VEC + l] = xi[l]
        return carry

    lax.fori_loop(0, BPW // VEC, _stage, 0)

    def _fire(c, ugrp, igrp, sem):
        cbase = c * K

        def body(j, carry):
            gu = pl.multiple_of((us_s[cbase + j] >> 3) << 3, 8)
            gi = pl.multiple_of((is_s[cbase + j] >> 3) << 3, 8)
            pltpu.async_copy(tuid_hbm.at[pl.ds(gu, 8)],
                             ugrp.at[pl.ds(j * 8, 8)], sem)
            pltpu.async_copy(tiid_hbm.at[pl.ds(gi, 8)],
                             igrp.at[pl.ds(j * 8, 8)], sem)
            return carry

        lax.fori_loop(0, K, body, 0)

    def _drain(ugrp, igrp, sem):
        pltpu.make_async_copy(tuid_hbm.at[pl.ds(0, K * 8)], ugrp,
                              sem).wait()
        pltpu.make_async_copy(tiid_hbm.at[pl.ds(0, K * 8)], igrp,
                              sem).wait()

    def _extract(c, ugrp, igrp, crows):
        cbase = c * K

        def body(j, carry):
            su = j * 8 + (us_s[cbase + j] & 7)
            si = j * 8 + (is_s[cbase + j] & 7)
            for k in range(D // VEC):
                crows[j, pl.ds(k * VEC, VEC)] = (
                    ugrp[su, pl.ds(k * VEC, VEC)])
                crows[j, pl.ds(D + k * VEC, VEC)] = (
                    igrp[si, pl.ds(k * VEC, VEC)])
            return carry

        lax.fori_loop(0, K, body, 0)

    def _wdesc(crows, sem_w, c):
        off = pl.multiple_of(base + c * K, 8)
        return pltpu.make_async_copy(
            crows, out_hbm.at[pl.ds(off, K)], sem_w)

    _fire(0, ugrp_a, igrp_a, sem_a)

    def _pair(c, carry):
        e = 2 * c

        @pl.when(c > 0)
        def _():
            _wdesc(crows_a, sem_wa, 0).wait()

        _fire(e + 1, ugrp_b, igrp_b, sem_b)
        _drain(ugrp_a, igrp_a, sem_a)
        _extract(e, ugrp_a, igrp_a, crows_a)
        _wdesc(crows_a, sem_wa, e).start()

        @pl.when(c > 0)
        def _():
            _wdesc(crows_b, sem_wb, 0).wait()

        @pl.when(c + 1 < NCHUNK // 2)
        def _():
            _fire(e + 2, ugrp_a, igrp_a, sem_a)

        _drain(ugrp_b, igrp_b, sem_b)
        _extract(e + 1, ugrp_b, igrp_b, crows_b)
        _wdesc(crows_b, sem_wb, e + 1).start()
        return carry

    lax.fori_loop(0, NCHUNK // 2, _pair, 0)
    _wdesc(crows_a, sem_wa, 0).wait()
    _wdesc(crows_b, sem_wb, 0).wait()


def kernel(uid, iid, table_uid, table_iid):
    uid = uid.astype(jnp.int32)
    iid = iid.astype(jnp.int32)
    return _emb_kernel(uid, iid, table_uid, table_iid)
